# baseline (speedup 1.0000x reference)
import functools

import jax
import jax.numpy as jnp
from jax import lax
from jax.experimental import pallas as pl
from jax.experimental.pallas import tpu as pltpu
from jax.experimental.pallas import tpu_sc as plsc

_VOCAB = 100000
_EMB = 64
_NOISE = 100
_KPAD = 128
_LANES = 128


def _sc_gather(tgt_flat, nidx_pad, embs, lpn):
    n = tgt_flat.shape[0]
    info = plsc.get_sparse_core_info()
    num_workers = info.num_cores * info.num_subcores
    per_w = n // num_workers
    chunks = per_w // _LANES
    group = 10
    groups = chunks // group
    grows = group * _LANES
    mesh = plsc.VectorSubcoreMesh(core_axis_name="c", subcore_axis_name="s")

    @functools.partial(
        pl.kernel,
        mesh=mesh,
        compiler_params=pltpu.CompilerParams(use_tc_tiling_on_sc=False),
        out_type=(
            jax.ShapeDtypeStruct((n, _EMB), jnp.float32),
            jax.ShapeDtypeStruct((n,), jnp.float32),
            jax.ShapeDtypeStruct((_KPAD, _EMB), jnp.float32),
            jax.ShapeDtypeStruct((_KPAD,), jnp.float32),
        ),
        scratch_types=(
            pltpu.VMEM((per_w,), jnp.int32),
            pltpu.VMEM((grows, _EMB), jnp.float32),
            pltpu.VMEM((grows,), jnp.float32),
            pltpu.VMEM((_KPAD,), jnp.int32),
            pltpu.SemaphoreType.DMA,
            pltpu.SemaphoreType.DMA,
        ),
    )
    def k(tgt_hbm, nidx_hbm, embs_hbm, lpn_hbm,
          tb_hbm, qt_hbm, nb_hbm, qn_hbm,
          idx_v, rows_v, qt_v, nidx_v, sem_r, sem_q):
        wid = lax.axis_index("s") * info.num_cores + lax.axis_index("c")
        base = wid * per_w
        pltpu.sync_copy(tgt_hbm.at[pl.ds(base, per_w)], idx_v)

        def body(g, carry):
            g0 = g * grows
            handles = []
            for c in range(group):
                idx = idx_v.at[pl.ds(g0 + c * _LANES, _LANES)]
                dst = rows_v.at[pl.ds(c * _LANES, _LANES)]
                handles.append(pltpu.async_copy(embs_hbm.at[idx], dst, sem_r))
                qdst = qt_v.at[pl.ds(c * _LANES, _LANES)]
                handles.append(pltpu.async_copy(lpn_hbm.at[idx], qdst, sem_q))
            for h in handles:
                h.wait()
            pltpu.sync_copy(rows_v, tb_hbm.at[pl.ds(base + g0, grows)])
            pltpu.sync_copy(qt_v, qt_hbm.at[pl.ds(base + g0, grows)])
            return carry

        lax.fori_loop(0, groups, body, 0)

        @pl.when(wid == 0)
        def _():
            pltpu.sync_copy(nidx_hbm, nidx_v)
            nrows = rows_v.at[pl.ds(0, _KPAD)]
            pltpu.async_copy(embs_hbm.at[nidx_v], nrows, sem_r).wait()
            pltpu.sync_copy(nrows, nb_hbm)
            nqt = qt_v.at[pl.ds(0, _KPAD)]
            pltpu.async_copy(lpn_hbm.at[nidx_v], nqt, sem_q).wait()
            pltpu.sync_copy(nqt, qn_hbm)

    return k(tgt_flat, nidx_pad, embs, lpn)


def _tc_loss(inp, tb3, qt2, nbz, e_last, qn_aug):
    batch, max_len, _ = inp.shape
    n = batch * max_len
    nblk = 32
    bb = batch // nblk

    rows = bb * max_len

    def body(inp_ref, tb_ref, qt_ref, nb_ref, e_ref, qn_ref, out_ref):
        i = pl.program_id(0)
        inp2 = inp_ref[...].reshape(rows, _EMB)
        tb = tb_ref[...]
        qt = qt_ref[...]
        nbv = nb_ref[...]
        ev = e_ref[...]
        qn = qn_ref[...]
        dn = (((1,), (1,)), ((), ()))
        scores = lax.dot_general(inp2, nbv, dn,
                                 preferred_element_type=jnp.float32)
        tsf = lax.dot_general(inp2 * tb, ev, dn,
                              preferred_element_type=jnp.float32)
        s_all = scores + tsf
        l0 = s_all[:, _KPAD - 1:] - qt
        ln = s_all - qn
        m = jnp.maximum(jnp.max(ln, axis=1, keepdims=True), l0)
        ssum = jnp.sum(jnp.exp(ln - m), axis=1, keepdims=True) + \
            jnp.exp(l0 - m)
        lse = m + jnp.log(ssum)
        part = jnp.sum(lse - l0, keepdims=True).reshape(1, 1)

        @pl.when(i == 0)
        def _():
            out_ref[...] = jnp.zeros_like(out_ref)

        out_ref[...] += part

    out = pl.pallas_call(
        body,
        grid=(nblk,),
        in_specs=[
            pl.BlockSpec((bb, max_len, _EMB), lambda i: (i, 0, 0)),
            pl.BlockSpec((rows, _EMB), lambda i: (i, 0)),
            pl.BlockSpec((rows, 1), lambda i: (i, 0)),
            pl.BlockSpec((_KPAD, _EMB), lambda i: (0, 0)),
            pl.BlockSpec((_KPAD, _EMB), lambda i: (0, 0)),
            pl.BlockSpec((1, _KPAD), lambda i: (0, 0)),
        ],
        out_specs=pl.BlockSpec((1, 1), lambda i: (0, 0)),
        out_shape=jax.ShapeDtypeStruct((1, 1), jnp.float32),
    )(inp, tb3, qt2, nbz, e_last, qn_aug)
    return out[0, 0] / n


def kernel(target, input, embs, logprob_noise):
    batch, max_len = target.shape
    n = batch * max_len
    nidx = jax.random.randint(jax.random.key(42), (1, 1, _NOISE), 0, _VOCAB,
                              dtype=jnp.int32)[0, 0]
    nidx_pad = jnp.concatenate(
        [nidx, jnp.zeros((_KPAD - _NOISE,), jnp.int32)])
    tgt_flat = target.reshape(n)
    tb2, qt1, nb, qn = _sc_gather(tgt_flat, nidx_pad, embs, logprob_noise)
    tb3 = tb2
    qt2 = qt1.reshape(n, 1)
    j = jnp.arange(_KPAD)
    nbz = jnp.where((j < _NOISE)[:, None], nb, 0.0)
    e_last = jnp.where((j == _KPAD - 1)[:, None],
                       jnp.ones((_KPAD, _EMB), jnp.float32), 0.0)
    qn_aug = jnp.where(j < _NOISE, qn, 1e30).reshape(1, _KPAD)
    return _tc_loss(input, tb3, qt2, nbz, e_last, qn_aug)
